# baseline (speedup 1.0000x reference)
import functools

import jax
import jax.numpy as jnp
from jax import lax
from jax.experimental import pallas as pl
from jax.experimental.pallas import tpu as pltpu
from jax.experimental.pallas import tpu_sc as plsc


@functools.lru_cache(maxsize=None)
def _make_sc_gather(B, N, D, bpw, nw, lanes):
    mesh = plsc.VectorSubcoreMesh(core_axis_name="c", subcore_axis_name="s")
    n_cores = 2

    @functools.partial(
        pl.kernel,
        mesh=mesh,
        out_type=jax.ShapeDtypeStruct((B, D), jnp.float32),
        scratch_types=[
            pltpu.VMEM((bpw,), jnp.int32),
            pltpu.VMEM((bpw, D), jnp.float32),
            pltpu.SemaphoreType.DMA,
        ],
    )
    def sc_gather(table_hbm, idx_hbm, out_hbm, idx_v, rows_v, sem):
        wid = lax.axis_index("s") * n_cores + lax.axis_index("c")
        base = wid * bpw
        pltpu.sync_copy(idx_hbm.at[pl.ds(base, bpw)], idx_v)
        lane = lax.iota(jnp.int32, 16)
        for j in range(bpw // lanes):
            i_vec = base + j * lanes + lane
            idx_v[pl.ds(j * lanes, lanes)] = (
                i_vec * N + idx_v[pl.ds(j * lanes, lanes)]
            )
        pltpu.async_copy(table_hbm.at[idx_v], rows_v, sem).wait()
        pltpu.sync_copy(rows_v, out_hbm.at[pl.ds(base, bpw)])

    return sc_gather


def _mean_body(inv_n, emb_ref, prev_ref, out_ref):
    d = prev_ref.shape[-1]
    out_ref[:, :d] = prev_ref[...]
    out_ref[:, d:] = jnp.sum(emb_ref[...], axis=1) * inv_n


@functools.lru_cache(maxsize=None)
def _make_tc_mean(B, N, D, bb):
    return pl.pallas_call(
        functools.partial(_mean_body, 1.0 / N),
        grid=(B // bb,),
        in_specs=[
            pl.BlockSpec((bb, N, D), lambda i: (i, 0, 0)),
            pl.BlockSpec((bb, D), lambda i: (i, 0)),
        ],
        out_specs=pl.BlockSpec((bb, 2 * D), lambda i: (i, 0)),
        out_shape=jax.ShapeDtypeStruct((B, 2 * D), jnp.float32),
        compiler_params=pltpu.CompilerParams(
            dimension_semantics=("arbitrary",),
        ),
    )


def kernel(embeddings, current_node):
    B, N, D = embeddings.shape
    nw, lanes = 32, 16
    bpw = B // nw

    table = embeddings.reshape(B * N, D)
    idx = current_node.reshape(B).astype(jnp.int32)

    prev = _make_sc_gather(B, N, D, bpw, nw, lanes)(table, idx)
    out = _make_tc_mean(B, N, D, 32)(embeddings, prev)
    return out.reshape(B, 1, 2 * D)


# speedup vs baseline: 1.1427x; 1.1427x over previous
import functools

import jax
import jax.numpy as jnp
from jax import lax
from jax.experimental import pallas as pl
from jax.experimental.pallas import tpu as pltpu
from jax.experimental.pallas import tpu_sc as plsc


@functools.lru_cache(maxsize=None)
def _make_sc_gather(B, N, D, bpw, nw, lanes):
    mesh = plsc.VectorSubcoreMesh(core_axis_name="c", subcore_axis_name="s")
    n_cores = 2

    @functools.partial(
        pl.kernel,
        mesh=mesh,
        out_type=jax.ShapeDtypeStruct((B, D), jnp.float32),
        scratch_types=[
            pltpu.VMEM((bpw,), jnp.int32),
            pltpu.VMEM((bpw, D), jnp.float32),
            pltpu.SemaphoreType.DMA,
        ],
    )
    def sc_gather(table_hbm, idx_hbm, out_hbm, idx_v, rows_v, sem):
        wid = lax.axis_index("s") * n_cores + lax.axis_index("c")
        base = wid * bpw
        pltpu.sync_copy(idx_hbm.at[pl.ds(base, bpw)], idx_v)
        lane = lax.iota(jnp.int32, 16)
        for j in range(bpw // lanes):
            i_vec = base + j * lanes + lane
            idx_v[pl.ds(j * lanes, lanes)] = (
                i_vec * N + idx_v[pl.ds(j * lanes, lanes)]
            )
        pltpu.async_copy(table_hbm.at[idx_v], rows_v, sem).wait()
        pltpu.sync_copy(rows_v, out_hbm.at[pl.ds(base, bpw)])

    return sc_gather


def _mean_body(inv_n, emb_ref, prev_ref, out_ref):
    d = prev_ref.shape[-1]
    out_ref[:, :d] = prev_ref[...]
    out_ref[:, d:] = jnp.sum(emb_ref[...], axis=1) * inv_n


@functools.lru_cache(maxsize=None)
def _make_tc_mean(B, N, D, bb):
    return pl.pallas_call(
        functools.partial(_mean_body, 1.0 / N),
        grid=(B // bb,),
        in_specs=[
            pl.BlockSpec((bb, N, D), lambda i: (i, 0, 0)),
            pl.BlockSpec((bb, D), lambda i: (i, 0)),
        ],
        out_specs=pl.BlockSpec((bb, 2 * D), lambda i: (i, 0)),
        out_shape=jax.ShapeDtypeStruct((B, 2 * D), jnp.float32),
        compiler_params=pltpu.CompilerParams(
            dimension_semantics=("arbitrary",),
        ),
    )


def kernel(embeddings, current_node):
    B, N, D = embeddings.shape
    nw, lanes = 32, 16
    bpw = B // nw

    table = embeddings.reshape(B * N, D)
    idx = current_node.reshape(B).astype(jnp.int32)

    prev = _make_sc_gather(B, N, D, bpw, nw, lanes)(table, idx)
    out = _make_tc_mean(B, N, D, 64)(embeddings, prev)
    return out.reshape(B, 1, 2 * D)


# speedup vs baseline: 1.1480x; 1.0047x over previous
import functools

import jax
import jax.numpy as jnp
from jax import lax
from jax.experimental import pallas as pl
from jax.experimental.pallas import tpu as pltpu
from jax.experimental.pallas import tpu_sc as plsc


@functools.lru_cache(maxsize=None)
def _make_sc_gather(B, N, D, bpw, nw, lanes):
    mesh = plsc.VectorSubcoreMesh(core_axis_name="c", subcore_axis_name="s")
    n_cores = 2

    @functools.partial(
        pl.kernel,
        mesh=mesh,
        out_type=jax.ShapeDtypeStruct((B, D), jnp.float32),
        scratch_types=[
            pltpu.VMEM((bpw,), jnp.int32),
            pltpu.VMEM((bpw, D), jnp.float32),
            pltpu.SemaphoreType.DMA,
        ],
    )
    def sc_gather(table_hbm, idx_hbm, out_hbm, idx_v, rows_v, sem):
        wid = lax.axis_index("s") * n_cores + lax.axis_index("c")
        base = wid * bpw
        pltpu.sync_copy(idx_hbm.at[pl.ds(base, bpw)], idx_v)
        lane = lax.iota(jnp.int32, 16)
        for j in range(bpw // lanes):
            i_vec = base + j * lanes + lane
            idx_v[pl.ds(j * lanes, lanes)] = (
                i_vec * N + idx_v[pl.ds(j * lanes, lanes)]
            )
        pltpu.async_copy(table_hbm.at[idx_v], rows_v, sem).wait()
        pltpu.sync_copy(rows_v, out_hbm.at[pl.ds(base, bpw)])

    return sc_gather


def _mean_body(inv_n, emb_ref, out_ref):
    out_ref[...] = jnp.sum(emb_ref[...], axis=1) * inv_n


@functools.lru_cache(maxsize=None)
def _make_tc_mean(B, N, D, bb):
    return pl.pallas_call(
        functools.partial(_mean_body, 1.0 / N),
        grid=(B // bb,),
        in_specs=[pl.BlockSpec((bb, N, D), lambda i: (i, 0, 0))],
        out_specs=pl.BlockSpec((bb, D), lambda i: (i, 0)),
        out_shape=jax.ShapeDtypeStruct((B, D), jnp.float32),
        compiler_params=pltpu.CompilerParams(
            dimension_semantics=("arbitrary",),
        ),
    )


def _combine_body(prev_ref, mean_ref, out_ref):
    d = prev_ref.shape[-1]
    out_ref[:, :d] = prev_ref[...]
    out_ref[:, d:] = mean_ref[...]


@functools.lru_cache(maxsize=None)
def _make_tc_combine(B, D, bb):
    return pl.pallas_call(
        _combine_body,
        grid=(B // bb,),
        in_specs=[
            pl.BlockSpec((bb, D), lambda i: (i, 0)),
            pl.BlockSpec((bb, D), lambda i: (i, 0)),
        ],
        out_specs=pl.BlockSpec((bb, 2 * D), lambda i: (i, 0)),
        out_shape=jax.ShapeDtypeStruct((B, 2 * D), jnp.float32),
        compiler_params=pltpu.CompilerParams(
            dimension_semantics=("arbitrary",),
        ),
    )


def kernel(embeddings, current_node):
    B, N, D = embeddings.shape
    nw, lanes = 32, 16
    bpw = B // nw

    table = embeddings.reshape(B * N, D)
    idx = current_node.reshape(B).astype(jnp.int32)

    prev = _make_sc_gather(B, N, D, bpw, nw, lanes)(table, idx)
    mean = _make_tc_mean(B, N, D, 64)(embeddings)
    out = _make_tc_combine(B, D, 1024)(prev, mean)
    return out.reshape(B, 1, 2 * D)
